# Initial kernel scaffold; baseline (speedup 1.0000x reference)
import jax
import jax.numpy as jnp
from jax.experimental import pallas as pl


def kernel(x, edge_index, edge_label_index, params):
    raise NotImplementedError("write your pallas kernel here")


# speedup vs baseline: 4.9927x; 4.9927x over previous
import functools

import jax
import jax.numpy as jnp
import numpy as np
from jax import lax
from jax.experimental import pallas as pl
from jax.experimental.pallas import tpu as pltpu
from jax.experimental.pallas import tpu_sc as plsc

N = 10000
H = 128
NUM_LAYERS = 4

NC = 2
NS = 16
NW = NC * NS

B = 128
NB = 88
E_PAD = NW * NB * B

NR = 10240
RPT = NR // NS

ELB = 25
EL_PAD = NW * ELB * B

_f32 = jnp.float32
_ALPHA = np.float32(1.0 / (NUM_LAYERS + 1))


def _lane0():
    return lax.iota(jnp.int32, 16) == 0


_BFMASK = np.int32(np.uint32(0xFFFF0000).astype(np.int32))


def _rne_bf16(x16):
    u = plsc.bitcast(x16, jnp.int32)
    r = u + jnp.int32(0x7FFF) + (lax.shift_right_logical(u, 16) & 1)
    return plsc.bitcast(r & _BFMASK, _f32)


RBLK = 2000
_GMM = N // RBLK


def _tc_mm_body(h_ref, wl_ref, bl_ref, wr_ref, br_ref, attc_ref,
                xl_ref, xr_ref, m_ref, umax, vmax):
    pid = pl.program_id(0)
    hb = h_ref[...].astype(jnp.bfloat16)
    xl = jnp.dot(hb, wl_ref[...].astype(jnp.bfloat16),
                 preferred_element_type=_f32) + bl_ref[...]
    xr = jnp.dot(hb, wr_ref[...].astype(jnp.bfloat16),
                 preferred_element_type=_f32) + br_ref[...]
    xl_ref[...] = xl
    xr_ref[...] = xr
    aa = jnp.abs(attc_ref[...]).astype(jnp.bfloat16)
    u = jnp.max(jnp.dot(jnp.abs(xl).astype(jnp.bfloat16), aa,
                        preferred_element_type=_f32))
    v = jnp.max(jnp.dot(jnp.abs(xr).astype(jnp.bfloat16), aa,
                        preferred_element_type=_f32))
    prev_u = jnp.where(pid == 0, jnp.float32(0.0), umax[0])
    prev_v = jnp.where(pid == 0, jnp.float32(0.0), vmax[0])
    uu = jnp.maximum(prev_u, u)
    vv = jnp.maximum(prev_v, v)
    umax[0] = uu
    vmax[0] = vv

    @pl.when(pid == _GMM - 1)
    def _():
        m_ref[...] = jnp.full((8, 128), uu + vv, _f32)


def _tc_mm(h, wl, bl, wr, br, attc):
    return pl.pallas_call(
        _tc_mm_body,
        grid=(_GMM,),
        in_specs=[
            pl.BlockSpec((RBLK, H), lambda i: (i, 0)),
            pl.BlockSpec((H, H), lambda i: (0, 0)),
            pl.BlockSpec((1, H), lambda i: (0, 0)),
            pl.BlockSpec((H, H), lambda i: (0, 0)),
            pl.BlockSpec((1, H), lambda i: (0, 0)),
            pl.BlockSpec((H, 1), lambda i: (0, 0)),
        ],
        out_specs=[
            pl.BlockSpec((RBLK, H), lambda i: (i, 0)),
            pl.BlockSpec((RBLK, H), lambda i: (i, 0)),
            pl.BlockSpec((8, 128), lambda i: (0, 0)),
        ],
        out_shape=[
            jax.ShapeDtypeStruct((N, H), _f32),
            jax.ShapeDtypeStruct((N, H), _f32),
            jax.ShapeDtypeStruct((8, 128), _f32),
        ],
        scratch_shapes=[pltpu.SMEM((1,), _f32), pltpu.SMEM((1,), _f32)],
    )(h, wl, bl, wr, br, attc)


def _tc_bn_body(outp0, outp1, den0, den1, bias, gamma, beta, acc_in,
                acc_ref, hbn_ref):
    num = outp0[0:N, :] + outp1[0:N, :]
    den = (den0[...] + den1[...]).reshape(NR)[0:N].reshape(N, 1)
    h = num / den + bias[...]
    mean = jnp.mean(h, axis=0, keepdims=True)
    var = jnp.mean(h * h, axis=0, keepdims=True) - mean * mean
    hbn = (h - mean) / jnp.sqrt(var + np.float32(1e-5)) * gamma[...] + beta[...]
    acc_ref[...] = acc_in[...] + _ALPHA * hbn
    hbn_ref[...] = hbn


def _tc_bn(outp0, outp1, den0, den1, bias, gamma, beta, acc):
    return pl.pallas_call(
        _tc_bn_body,
        compiler_params=pltpu.CompilerParams(vmem_limit_bytes=100 * 1024 * 1024),
        out_shape=[
            jax.ShapeDtypeStruct((N, H), _f32),
            jax.ShapeDtypeStruct((N, H), _f32),
        ],
    )(outp0, outp1, den0, den1, bias, gamma, beta, acc)


def _sc_edge_body(xl_hbm, xr_hbm, src_hbm, dst_hbm, att_hbm, m_hbm,
                  outp0, outp1, den0, den1,
                  src_g, dst_g, xl_buf, xr_buf, ex_buf, att_v, m_v,
                  dbuf, out_sh, den_sh, sem_l, sem_r):
    cid = lax.axis_index("c")
    sid = lax.axis_index("s")
    wid = sid * NC + cid

    pltpu.sync_copy(att_hbm, att_v)
    pltpu.sync_copy(m_hbm, m_v)

    z16 = jnp.zeros((16,), _f32)

    def _zrow(t, c):
        xl_buf[t // 8, pl.ds((t % 8) * 16, 16)] = z16
        return c

    lax.fori_loop(0, B * 8, _zrow, 0)

    def _zden(t, c):
        dbuf[pl.ds(t * 16, 16)] = z16
        return c

    lax.fori_loop(0, RPT // 16, _zden, 0)

    r0 = sid * RPT
    for t in range(RPT // B):
        pltpu.sync_copy(xl_buf, out_sh.at[pl.ds(r0 + t * B, B)])
    pltpu.sync_copy(dbuf, den_sh.at[pl.ds(r0, RPT)])
    plsc.subcore_barrier()

    mrow = m_v[0, pl.ds(0, 16)]
    lane0 = _lane0()
    atts = [_rne_bf16(att_v[pl.ds(k * 16, 16)]) for k in range(8)]

    def _blk(t, c):
        hl = pltpu.async_copy(xl_hbm.at[src_g.at[t]], xl_buf, sem_l)
        hr = pltpu.async_copy(xr_hbm.at[dst_g.at[t]], xr_buf, sem_r)
        hl.wait()
        hr.wait()

        @plsc.parallel_loop(0, B, step=1, unroll=2)
        def _edge(i):
            acc = jnp.zeros((16,), _f32)
            regs = []
            for k in range(8):
                a = xl_buf[i, pl.ds(k * 16, 16)]
                b = xr_buf[i, pl.ds(k * 16, 16)]
                z = a + b
                e = jnp.maximum(z, np.float32(0.2) * z)
                acc = acc + _rne_bf16(e) * atts[k]
                regs.append(a)
            s = jnp.sum(acc)
            ex16 = jnp.exp(jnp.full((16,), s, _f32) - mrow)
            for k in range(8):
                xl_buf[i, pl.ds(k * 16, 16)] = regs[k] * ex16
            plsc.store_scatter(ex_buf, [jnp.full((16,), i, jnp.int32)], ex16,
                               mask=lane0)

        pltpu.sync_copy(xl_buf, out_sh.at[dst_g.at[t]], add=True)
        pltpu.sync_copy(ex_buf, den_sh.at[dst_g.at[t]], add=True)
        return c

    def _grp(g, c):
        pltpu.sync_copy(src_hbm.at[pl.ds(wid * NB + g * 8, 8)], src_g)
        pltpu.sync_copy(dst_hbm.at[pl.ds(wid * NB + g * 8, 8)], dst_g)
        lax.fori_loop(0, 8, _blk, 0)
        return c

    lax.fori_loop(0, NB // 8, _grp, 0)
    plsc.subcore_barrier()

    pltpu.sync_copy(den_sh.at[pl.ds(r0, RPT)], dbuf)

    @pl.when(cid == 0)
    def _():
        pltpu.sync_copy(out_sh.at[pl.ds(r0, RPT)], outp0.at[pl.ds(r0, RPT)])
        pltpu.sync_copy(dbuf, den0.at[pl.ds(r0, RPT)])

    @pl.when(cid == 1)
    def _():
        pltpu.sync_copy(out_sh.at[pl.ds(r0, RPT)], outp1.at[pl.ds(r0, RPT)])
        pltpu.sync_copy(dbuf, den1.at[pl.ds(r0, RPT)])


def _sc_edge(xl, xr, src2d, dst2d, att, m):
    mesh = plsc.VectorSubcoreMesh(core_axis_name="c", subcore_axis_name="s")
    fn = pl.kernel(
        _sc_edge_body,
        out_type=[
            jax.ShapeDtypeStruct((NR, H), _f32),
            jax.ShapeDtypeStruct((NR, H), _f32),
            jax.ShapeDtypeStruct((NR,), _f32),
            jax.ShapeDtypeStruct((NR,), _f32),
        ],
        mesh=mesh,
        compiler_params=pltpu.CompilerParams(needs_layout_passes=False),
        scratch_types=[
            pltpu.VMEM((8, B), jnp.int32),
            pltpu.VMEM((8, B), jnp.int32),
            pltpu.VMEM((B, H), _f32),
            pltpu.VMEM((B, H), _f32),
            pltpu.VMEM((B,), _f32),
            pltpu.VMEM((H,), _f32),
            pltpu.VMEM((8, 128), _f32),
            pltpu.VMEM((RPT,), _f32),
            pltpu.VMEM_SHARED((NR, H), _f32),
            pltpu.VMEM_SHARED((NR,), _f32),
            pltpu.SemaphoreType.DMA,
            pltpu.SemaphoreType.DMA,
        ],
    )
    return fn(xl, xr, src2d, dst2d, att, m)


def _sc_dot_body(acc_hbm, ia_hbm, ib_hbm, out_hbm,
                 ia, ib, a_buf, b_buf, p_buf, sem_a, sem_b):
    cid = lax.axis_index("c")
    sid = lax.axis_index("s")
    wid = sid * NC + cid

    pltpu.sync_copy(ia_hbm.at[pl.ds(wid * ELB * B, ELB * B)], ia)
    pltpu.sync_copy(ib_hbm.at[pl.ds(wid * ELB * B, ELB * B)], ib)
    lane0 = _lane0()

    def _blk(j, c):
        ha = pltpu.async_copy(acc_hbm.at[ia.at[pl.ds(j * B, B)]], a_buf, sem_a)
        hb = pltpu.async_copy(acc_hbm.at[ib.at[pl.ds(j * B, B)]], b_buf, sem_b)
        ha.wait()
        hb.wait()

        @plsc.parallel_loop(0, B, step=1, unroll=2)
        def _edge(i):
            acc = jnp.zeros((16,), _f32)
            for k in range(8):
                acc = acc + (a_buf[i, pl.ds(k * 16, 16)]
                             * b_buf[i, pl.ds(k * 16, 16)])
            s16 = jnp.full((16,), jnp.sum(acc), _f32)
            plsc.store_scatter(p_buf, [jnp.full((16,), i, jnp.int32)], s16,
                               mask=lane0)

        pltpu.sync_copy(p_buf, out_hbm.at[pl.ds(wid * ELB * B + j * B, B)])
        return c

    lax.fori_loop(0, ELB, _blk, 0)


def _sc_dot(acc, ia2d, ib2d):
    mesh = plsc.VectorSubcoreMesh(core_axis_name="c", subcore_axis_name="s")
    fn = pl.kernel(
        _sc_dot_body,
        out_type=jax.ShapeDtypeStruct((EL_PAD,), _f32),
        mesh=mesh,
        compiler_params=pltpu.CompilerParams(needs_layout_passes=False),
        scratch_types=[
            pltpu.VMEM((ELB * B,), jnp.int32),
            pltpu.VMEM((ELB * B,), jnp.int32),
            pltpu.VMEM((B, H), _f32),
            pltpu.VMEM((B, H), _f32),
            pltpu.VMEM((B,), _f32),
            pltpu.SemaphoreType.DMA,
            pltpu.SemaphoreType.DMA,
        ],
    )
    return fn(acc, ia2d, ib2d)


def kernel(x, edge_index, edge_label_index, params):
    i32 = jnp.int32
    loop = jnp.arange(N, dtype=i32)
    e_tot = edge_index.shape[1] + N
    src = jnp.concatenate(
        [edge_index[0], loop, jnp.zeros((E_PAD - e_tot,), i32)])
    dst = jnp.concatenate(
        [edge_index[1], loop, jnp.full((E_PAD - e_tot,), N, i32)])
    src2d = src.reshape(NW * NB, B)
    dst2d = dst.reshape(NW * NB, B)

    el = edge_label_index.shape[1]
    ia = jnp.concatenate(
        [edge_label_index[0], jnp.zeros((EL_PAD - el,), i32)])
    ib = jnp.concatenate(
        [edge_label_index[1], jnp.zeros((EL_PAD - el,), i32)])

    def prep(p):
        return (p['Wl'], p['bl'].reshape(1, H), p['Wr'], p['br'].reshape(1, H),
                p['att'].reshape(H, 1), p['att'])

    wl, bl, wr, br, attc, att = prep(params[0])
    xl, xr, m = _tc_mm(x, wl, bl, wr, br, attc)
    acc = jnp.zeros((N, H), _f32)
    for i in range(NUM_LAYERS):
        outp0, outp1, den0, den1 = _sc_edge(xl, xr, src2d, dst2d, att, m)
        den0 = den0.reshape(NR // 128, 128)
        den1 = den1.reshape(NR // 128, 128)
        p = params[i]
        bias = p['bias'].reshape(1, H)
        gamma = p['gamma'].reshape(1, H)
        beta = p['beta'].reshape(1, H)
        acc, hbn = _tc_bn(outp0, outp1, den0, den1, bias, gamma, beta, acc)
        if i + 1 < NUM_LAYERS:
            wl, bl, wr, br, attc, att = prep(params[i + 1])
            xl, xr, m = _tc_mm(hbn, wl, bl, wr, br, attc)

    preds = _sc_dot(acc, ia, ib)
    return preds[0:el]


# speedup vs baseline: 4.9987x; 1.0012x over previous
import functools

import jax
import jax.numpy as jnp
import numpy as np
from jax import lax
from jax.experimental import pallas as pl
from jax.experimental.pallas import tpu as pltpu
from jax.experimental.pallas import tpu_sc as plsc

N = 10000
H = 128
NUM_LAYERS = 4

NC = 2
NS = 16
NW = NC * NS

B = 128
NB = 88
E_PAD = NW * NB * B
DB = 128

NR = 10240
RPT = NR // NS

ELB = 25
EL_PAD = NW * ELB * DB

_f32 = jnp.float32
_ALPHA = np.float32(1.0 / (NUM_LAYERS + 1))


def _lane0():
    return lax.iota(jnp.int32, 16) == 0


_BFMASK = np.int32(np.uint32(0xFFFF0000).astype(np.int32))


def _rne_bf16(x16):
    u = plsc.bitcast(x16, jnp.int32)
    r = u + jnp.int32(0x7FFF) + (lax.shift_right_logical(u, 16) & 1)
    return plsc.bitcast(r & _BFMASK, _f32)


RBLK = 2000
_GMM = N // RBLK


def _tc_mm_body(h_ref, wl_ref, bl_ref, wr_ref, br_ref, attc_ref,
                xl_ref, xr_ref, m_ref, umax, vmax):
    pid = pl.program_id(0)
    hb = h_ref[...].astype(jnp.bfloat16)
    xl = jnp.dot(hb, wl_ref[...].astype(jnp.bfloat16),
                 preferred_element_type=_f32) + bl_ref[...]
    xr = jnp.dot(hb, wr_ref[...].astype(jnp.bfloat16),
                 preferred_element_type=_f32) + br_ref[...]
    xl_ref[...] = xl
    xr_ref[...] = xr
    aa = jnp.abs(attc_ref[...]).astype(jnp.bfloat16)
    u = jnp.max(jnp.dot(jnp.abs(xl).astype(jnp.bfloat16), aa,
                        preferred_element_type=_f32))
    v = jnp.max(jnp.dot(jnp.abs(xr).astype(jnp.bfloat16), aa,
                        preferred_element_type=_f32))
    prev_u = jnp.where(pid == 0, jnp.float32(0.0), umax[0])
    prev_v = jnp.where(pid == 0, jnp.float32(0.0), vmax[0])
    uu = jnp.maximum(prev_u, u)
    vv = jnp.maximum(prev_v, v)
    umax[0] = uu
    vmax[0] = vv

    @pl.when(pid == _GMM - 1)
    def _():
        m_ref[...] = jnp.full((8, 128), uu + vv, _f32)


def _tc_mm(h, wl, bl, wr, br, attc):
    return pl.pallas_call(
        _tc_mm_body,
        grid=(_GMM,),
        in_specs=[
            pl.BlockSpec((RBLK, H), lambda i: (i, 0)),
            pl.BlockSpec((H, H), lambda i: (0, 0)),
            pl.BlockSpec((1, H), lambda i: (0, 0)),
            pl.BlockSpec((H, H), lambda i: (0, 0)),
            pl.BlockSpec((1, H), lambda i: (0, 0)),
            pl.BlockSpec((H, 1), lambda i: (0, 0)),
        ],
        out_specs=[
            pl.BlockSpec((RBLK, H), lambda i: (i, 0)),
            pl.BlockSpec((RBLK, H), lambda i: (i, 0)),
            pl.BlockSpec((8, 128), lambda i: (0, 0)),
        ],
        out_shape=[
            jax.ShapeDtypeStruct((N, H), _f32),
            jax.ShapeDtypeStruct((N, H), _f32),
            jax.ShapeDtypeStruct((8, 128), _f32),
        ],
        scratch_shapes=[pltpu.SMEM((1,), _f32), pltpu.SMEM((1,), _f32)],
    )(h, wl, bl, wr, br, attc)


def _tc_bn_body(outp0, outp1, den0, den1, bias, gamma, beta, acc_in,
                acc_ref, hbn_ref):
    num = outp0[0:N, :] + outp1[0:N, :]
    den = (den0[...] + den1[...]).reshape(NR)[0:N].reshape(N, 1)
    h = num / den + bias[...]
    mean = jnp.mean(h, axis=0, keepdims=True)
    var = jnp.mean(h * h, axis=0, keepdims=True) - mean * mean
    hbn = (h - mean) / jnp.sqrt(var + np.float32(1e-5)) * gamma[...] + beta[...]
    acc_ref[...] = acc_in[...] + _ALPHA * hbn
    hbn_ref[...] = hbn


def _tc_bn(outp0, outp1, den0, den1, bias, gamma, beta, acc):
    return pl.pallas_call(
        _tc_bn_body,
        compiler_params=pltpu.CompilerParams(vmem_limit_bytes=100 * 1024 * 1024),
        out_shape=[
            jax.ShapeDtypeStruct((N, H), _f32),
            jax.ShapeDtypeStruct((N, H), _f32),
        ],
    )(outp0, outp1, den0, den1, bias, gamma, beta, acc)


def _sc_edge_body(xl_hbm, xr_hbm, src_hbm, dst_hbm, att_hbm, m_hbm,
                  outp0, outp1, den0, den1,
                  src_g, dst_g, xl_buf, xr_buf, ex_buf,
                  att_v, m_v, dbuf, out_sh, den_sh, sem_l, sem_r):
    cid = lax.axis_index("c")
    sid = lax.axis_index("s")
    wid = sid * NC + cid

    pltpu.sync_copy(att_hbm, att_v)
    pltpu.sync_copy(m_hbm, m_v)

    z16 = jnp.zeros((16,), _f32)

    def _zrow(t, c):
        xl_buf[t // 8, pl.ds((t % 8) * 16, 16)] = z16
        return c

    lax.fori_loop(0, B * 8, _zrow, 0)

    def _zden(t, c):
        dbuf[pl.ds(t * 16, 16)] = z16
        return c

    lax.fori_loop(0, RPT // 16, _zden, 0)

    r0 = sid * RPT
    for t in range(RPT // B):
        pltpu.sync_copy(xl_buf, out_sh.at[pl.ds(r0 + t * B, B)])
    pltpu.sync_copy(dbuf, den_sh.at[pl.ds(r0, RPT)])
    plsc.subcore_barrier()

    mrow = m_v[0, pl.ds(0, 16)]
    lane0 = _lane0()
    atts = [_rne_bf16(att_v[pl.ds(k * 16, 16)]) for k in range(8)]

    def _blk(t, c):
        hl = pltpu.async_copy(xl_hbm.at[src_g.at[t]], xl_buf, sem_l)
        hr = pltpu.async_copy(xr_hbm.at[dst_g.at[t]], xr_buf, sem_r)
        hl.wait()
        hr.wait()

        @plsc.parallel_loop(0, B, step=1, unroll=2)
        def _edge(i):
            acc = jnp.zeros((16,), _f32)
            regs = []
            for k in range(8):
                a = xl_buf[i, pl.ds(k * 16, 16)]
                b = xr_buf[i, pl.ds(k * 16, 16)]
                z = a + b
                e = jnp.maximum(z, np.float32(0.2) * z)
                acc = acc + _rne_bf16(e) * atts[k]
                regs.append(a)
            s = jnp.sum(acc)
            ex16 = jnp.exp(jnp.full((16,), s, _f32) - mrow)
            for k in range(8):
                xl_buf[i, pl.ds(k * 16, 16)] = regs[k] * ex16
            plsc.store_scatter(ex_buf, [jnp.full((16,), i, jnp.int32)], ex16,
                               mask=lane0)

        pltpu.sync_copy(xl_buf, out_sh.at[dst_g.at[t]], add=True)
        pltpu.sync_copy(ex_buf, den_sh.at[dst_g.at[t]], add=True)
        return c

    def _grp(g, c):
        pltpu.sync_copy(src_hbm.at[pl.ds(wid * NB + g * 8, 8)], src_g)
        pltpu.sync_copy(dst_hbm.at[pl.ds(wid * NB + g * 8, 8)], dst_g)
        lax.fori_loop(0, 8, _blk, 0)
        return c

    lax.fori_loop(0, NB // 8, _grp, 0)
    plsc.subcore_barrier()

    pltpu.sync_copy(den_sh.at[pl.ds(r0, RPT)], dbuf)

    @pl.when(cid == 0)
    def _():
        pltpu.sync_copy(out_sh.at[pl.ds(r0, RPT)], outp0.at[pl.ds(r0, RPT)])
        pltpu.sync_copy(dbuf, den0.at[pl.ds(r0, RPT)])

    @pl.when(cid == 1)
    def _():
        pltpu.sync_copy(out_sh.at[pl.ds(r0, RPT)], outp1.at[pl.ds(r0, RPT)])
        pltpu.sync_copy(dbuf, den1.at[pl.ds(r0, RPT)])


def _sc_edge(xl, xr, src2d, dst2d, att, m):
    mesh = plsc.VectorSubcoreMesh(core_axis_name="c", subcore_axis_name="s")
    fn = pl.kernel(
        _sc_edge_body,
        out_type=[
            jax.ShapeDtypeStruct((NR, H), _f32),
            jax.ShapeDtypeStruct((NR, H), _f32),
            jax.ShapeDtypeStruct((NR,), _f32),
            jax.ShapeDtypeStruct((NR,), _f32),
        ],
        mesh=mesh,
        compiler_params=pltpu.CompilerParams(needs_layout_passes=False),
        scratch_types=[
            pltpu.VMEM((8, B), jnp.int32),
            pltpu.VMEM((8, B), jnp.int32),
            pltpu.VMEM((B, H), _f32),
            pltpu.VMEM((B, H), _f32),
            pltpu.VMEM((B,), _f32),
            pltpu.VMEM((H,), _f32),
            pltpu.VMEM((8, 128), _f32),
            pltpu.VMEM((RPT,), _f32),
            pltpu.VMEM_SHARED((NR, H), _f32),
            pltpu.VMEM_SHARED((NR,), _f32),
            pltpu.SemaphoreType.DMA,
            pltpu.SemaphoreType.DMA,
        ],
    )
    return fn(xl, xr, src2d, dst2d, att, m)


def _sc_dot_body(acc_hbm, ia_hbm, ib_hbm, out_hbm,
                 ia, ib, a_buf, b_buf, p_buf, sem_a, sem_b):
    cid = lax.axis_index("c")
    sid = lax.axis_index("s")
    wid = sid * NC + cid

    pltpu.sync_copy(ia_hbm.at[pl.ds(wid * ELB * DB, ELB * DB)], ia)
    pltpu.sync_copy(ib_hbm.at[pl.ds(wid * ELB * DB, ELB * DB)], ib)
    lane0 = _lane0()

    def _blk(j, c):
        ha = pltpu.async_copy(acc_hbm.at[ia.at[pl.ds(j * DB, DB)]], a_buf, sem_a)
        hb = pltpu.async_copy(acc_hbm.at[ib.at[pl.ds(j * DB, DB)]], b_buf, sem_b)
        ha.wait()
        hb.wait()

        @plsc.parallel_loop(0, DB, step=1, unroll=2)
        def _edge(i):
            acc = jnp.zeros((16,), _f32)
            for k in range(8):
                acc = acc + (a_buf[i, pl.ds(k * 16, 16)]
                             * b_buf[i, pl.ds(k * 16, 16)])
            s16 = jnp.full((16,), jnp.sum(acc), _f32)
            plsc.store_scatter(p_buf, [jnp.full((16,), i, jnp.int32)], s16,
                               mask=lane0)

        pltpu.sync_copy(p_buf, out_hbm.at[pl.ds(wid * ELB * B + j * DB, DB)])
        return c

    lax.fori_loop(0, ELB, _blk, 0)


def _sc_dot(acc, ia2d, ib2d):
    mesh = plsc.VectorSubcoreMesh(core_axis_name="c", subcore_axis_name="s")
    fn = pl.kernel(
        _sc_dot_body,
        out_type=jax.ShapeDtypeStruct((EL_PAD,), _f32),
        mesh=mesh,
        compiler_params=pltpu.CompilerParams(needs_layout_passes=False),
        scratch_types=[
            pltpu.VMEM((ELB * DB,), jnp.int32),
            pltpu.VMEM((ELB * DB,), jnp.int32),
            pltpu.VMEM((DB, H), _f32),
            pltpu.VMEM((DB, H), _f32),
            pltpu.VMEM((DB,), _f32),
            pltpu.SemaphoreType.DMA,
            pltpu.SemaphoreType.DMA,
        ],
    )
    return fn(acc, ia2d, ib2d)


def kernel(x, edge_index, edge_label_index, params):
    i32 = jnp.int32
    loop = jnp.arange(N, dtype=i32)
    e_tot = edge_index.shape[1] + N
    src = jnp.concatenate(
        [edge_index[0], loop, jnp.zeros((E_PAD - e_tot,), i32)])
    dst = jnp.concatenate(
        [edge_index[1], loop, jnp.full((E_PAD - e_tot,), N, i32)])
    src2d = src.reshape(NW * NB, B)
    dst2d = dst.reshape(NW * NB, B)

    el = edge_label_index.shape[1]
    ia = jnp.concatenate(
        [edge_label_index[0], jnp.zeros((EL_PAD - el,), i32)])
    ib = jnp.concatenate(
        [edge_label_index[1], jnp.zeros((EL_PAD - el,), i32)])

    def prep(p):
        return (p['Wl'], p['bl'].reshape(1, H), p['Wr'], p['br'].reshape(1, H),
                p['att'].reshape(H, 1), p['att'])

    wl, bl, wr, br, attc, att = prep(params[0])
    xl, xr, m = _tc_mm(x, wl, bl, wr, br, attc)
    acc = jnp.zeros((N, H), _f32)
    for i in range(NUM_LAYERS):
        outp0, outp1, den0, den1 = _sc_edge(xl, xr, src2d, dst2d, att, m)
        den0 = den0.reshape(NR // 128, 128)
        den1 = den1.reshape(NR // 128, 128)
        p = params[i]
        bias = p['bias'].reshape(1, H)
        gamma = p['gamma'].reshape(1, H)
        beta = p['beta'].reshape(1, H)
        acc, hbn = _tc_bn(outp0, outp1, den0, den1, bias, gamma, beta, acc)
        if i + 1 < NUM_LAYERS:
            wl, bl, wr, br, attc, att = prep(params[i + 1])
            xl, xr, m = _tc_mm(hbn, wl, bl, wr, br, attc)

    preds = _sc_dot(acc, ia, ib)
    return preds[0:el]
